# baseline (speedup 1.0000x reference)
import functools

import jax
import jax.numpy as jnp
from jax import lax
from jax.experimental import pallas as pl
from jax.experimental.pallas import tpu as pltpu
from jax.experimental.pallas import tpu_sc as plsc

VOCAB = 1000000
EMBED = 64
BATCH = 4096
SEQ = 200

_NC = 2
_NS = 16
_NW = _NC * _NS
_BPW = BATCH // _NW


def _make_emb():
    mesh = plsc.VectorSubcoreMesh(core_axis_name="c", subcore_axis_name="s")

    @functools.partial(
        pl.kernel,
        mesh=mesh,
        out_type=jax.ShapeDtypeStruct((SEQ, EMBED, BATCH), jnp.float32),
        compiler_params=pltpu.CompilerParams(
            use_tc_tiling_on_sc=False, needs_layout_passes=False),
        scratch_types=[
            pltpu.VMEM((SEQ, _BPW), jnp.int32),
            pltpu.VMEM((_BPW, EMBED), jnp.float32),
            pltpu.VMEM((_BPW, EMBED), jnp.float32),
            pltpu.VMEM((EMBED, _BPW), jnp.float32),
            pltpu.VMEM((EMBED, _BPW), jnp.float32),
            pltpu.SemaphoreType.DMA,
            pltpu.SemaphoreType.DMA,
        ],
    )
    def emb(idx_hbm, table_hbm, out_hbm, idx_v, gb0, gb1, tb0, tb1,
            gsem, wsem):
        wid = lax.axis_index("s") * _NC + lax.axis_index("c")
        b0 = wid * _BPW
        pltpu.sync_copy(idx_hbm.at[:, pl.ds(b0, _BPW)], idx_v)

        def gstart(s, gb):
            pltpu.make_async_copy(
                table_hbm.at[idx_v.at[s, :]], gb, gsem).start()

        def gwait(gb):
            pltpu.make_async_copy(
                table_hbm.at[idx_v.at[0, :]], gb, gsem).wait()

        def wstart(s, tb):
            pltpu.make_async_copy(
                tb, out_hbm.at[s, :, pl.ds(b0, _BPW)], wsem).start()

        def wwait(tb):
            pltpu.make_async_copy(
                tb, out_hbm.at[0, :, pl.ds(b0, _BPW)], wsem).wait()

        i16 = lax.iota(jnp.int32, 16)

        def transpose(gb, tb):
            for e in range(EMBED):
                col = jnp.full((16,), e, jnp.int32)
                for g in range(_BPW // 16):
                    v = plsc.load_gather(gb, [i16 + 16 * g, col])
                    tb[e, pl.ds(16 * g, 16)] = v

        gstart(0, gb0)
        gstart(1, gb1)
        gwait(gb0)
        transpose(gb0, tb0)
        wstart(0, tb0)
        gstart(2, gb0)
        gwait(gb1)
        transpose(gb1, tb1)
        wstart(1, tb1)
        gstart(3, gb1)

        def body(j, carry):
            s0 = 2 * j + 2
            gwait(gb0)
            wwait(tb0)
            transpose(gb0, tb0)
            wstart(s0, tb0)
            gstart(s0 + 2, gb0)
            s1 = s0 + 1
            gwait(gb1)
            wwait(tb1)
            transpose(gb1, tb1)
            wstart(s1, tb1)
            gstart(s1 + 2, gb1)
            return carry

        lax.fori_loop(0, (SEQ - 4) // 2, body, 0)
        gwait(gb0)
        wwait(tb0)
        transpose(gb0, tb0)
        wstart(SEQ - 2, tb0)
        gwait(gb1)
        wwait(tb1)
        transpose(gb1, tb1)
        wstart(SEQ - 1, tb1)
        wwait(tb0)
        wwait(tb1)

    return emb


_emb = _make_emb()


def kernel(input, word_embed):
    idx_t = input.T.astype(jnp.int32)
    out_t = _emb(idx_t, word_embed)
    return out_t.transpose(2, 0, 1)


# speedup vs baseline: 1.7945x; 1.7945x over previous
import functools

import jax
import jax.numpy as jnp
from jax import lax
from jax.experimental import pallas as pl
from jax.experimental.pallas import tpu as pltpu
from jax.experimental.pallas import tpu_sc as plsc

VOCAB = 1000000
EMBED = 64
BATCH = 4096
SEQ = 200
NTOK = BATCH * SEQ

_NC = 2
_NS = 16
_NW = _NC * _NS
_RPW = BATCH // _NW
_BPW = _RPW * SEQ
_GRP = 2
_NG = _RPW // _GRP


def _make_emb():
    mesh = plsc.VectorSubcoreMesh(core_axis_name="c", subcore_axis_name="s")

    @functools.partial(
        pl.kernel,
        mesh=mesh,
        out_type=jax.ShapeDtypeStruct((BATCH, SEQ, EMBED), jnp.float32),
        compiler_params=pltpu.CompilerParams(
            use_tc_tiling_on_sc=False,
            disable_bounds_checks=True,
            disable_semaphore_checks=True),
        scratch_types=[
            pltpu.VMEM((_BPW,), jnp.int32),
            pltpu.VMEM((2 * _GRP, SEQ, EMBED), jnp.float32),
            pltpu.SemaphoreType.DMA,
            pltpu.SemaphoreType.DMA,
        ],
    )
    def emb(idx_hbm, table_hbm, out_hbm, idx_v, bufs, gsem, wsem):
        wid = lax.axis_index("s") * _NC + lax.axis_index("c")
        base = wid * _BPW
        row0 = wid * _RPW
        pltpu.sync_copy(idx_hbm.at[pl.ds(base, _BPW)], idx_v)

        def gstart(r, b):
            pltpu.make_async_copy(
                table_hbm.at[idx_v.at[pl.ds(r * SEQ, SEQ)]], bufs.at[b],
                gsem).start()

        def gwait(b):
            pltpu.make_async_copy(
                table_hbm.at[idx_v.at[pl.ds(0, SEQ)]], bufs.at[b],
                gsem).wait()

        def wstart(r, b):
            pltpu.make_async_copy(
                bufs.at[b], out_hbm.at[row0 + r], wsem).start()

        def wwait(b):
            pltpu.make_async_copy(
                bufs.at[b], out_hbm.at[row0], wsem).wait()

        def g_start(g, bb):
            for b in range(_GRP):
                gstart(g * _GRP + b, bb + b)

        def g_wait(bb):
            for b in range(_GRP):
                gwait(bb + b)

        def w_start(g, bb):
            for b in range(_GRP):
                wstart(g * _GRP + b, bb + b)

        def w_wait(bb):
            for b in range(_GRP):
                wwait(bb + b)

        g_start(0, 0)
        g_wait(0)
        w_start(0, 0)
        g_start(1, _GRP)

        def body(j, carry):
            g1 = 2 * j + 1
            g_wait(_GRP)
            w_wait(0)
            w_start(g1, _GRP)
            g_start(g1 + 1, 0)
            g2 = g1 + 1
            g_wait(0)
            w_wait(_GRP)
            w_start(g2, 0)
            g_start(g2 + 1, _GRP)
            return carry

        lax.fori_loop(0, (_NG - 2) // 2, body, 0)
        g_wait(_GRP)
        w_wait(0)
        w_start(_NG - 1, _GRP)
        w_wait(_GRP)

    return emb


_emb = _make_emb()


def kernel(input, word_embed):
    idx = input.reshape(-1).astype(jnp.int32)
    return _emb(idx, word_embed)
